# Initial kernel scaffold; baseline (speedup 1.0000x reference)
import jax
import jax.numpy as jnp
from jax.experimental import pallas as pl


def kernel(x, edge_index, batch, W_exp, b_exp, Ws, bs):
    raise NotImplementedError("write your pallas kernel here")


# speedup vs baseline: 4.6119x; 4.6119x over previous
import functools

import jax
import jax.numpy as jnp
from jax import lax
from jax.experimental import pallas as pl
from jax.experimental.pallas import tpu as pltpu
from jax.experimental.pallas import tpu_sc as plsc

N = 50000
E = 800000
D = 128
G = 512
NPAD = 51200
NQ = 4
DQ = D // NQ
ACCR = 53248
EPS = 51200
EPAD = 16 * EPS
CH = 128
NCH = EPS // CH
BN = 1024
GRID = NPAD // BN

_mesh = plsc.VectorSubcoreMesh(core_axis_name="c", subcore_axis_name="s")
_SC_PARAMS = pltpu.CompilerParams(use_tc_tiling_on_sc=False)


def _zero_vmem(ref, nwords):
    zero = jnp.zeros((16,), jnp.float32)

    def body(i, carry):
        ref[pl.ds(i * 16, 16)] = zero
        return carry

    lax.fori_loop(0, nwords // 16, body, 0)


def _zero_vmem2(ref, nrows, ncols):
    zero = jnp.zeros((16,), jnp.float32)

    def body(i, carry):
        for l in range(ncols // 16):
            ref[i, pl.ds(l * 16, 16)] = zero
        return carry

    lax.fori_loop(0, nrows, body, 0)


@functools.partial(
    pl.kernel,
    mesh=_mesh,
    out_type=jax.ShapeDtypeStruct((2, ACCR), jnp.float32),
    scratch_types=[
        pltpu.VMEM((CH,), jnp.int32),
        pltpu.VMEM((CH,), jnp.float32),
        pltpu.VMEM((3328,), jnp.float32),
        pltpu.VMEM_SHARED((ACCR,), jnp.float32),
    ],
    compiler_params=_SC_PARAMS,
)
def _deg_kernel(c_hbm, out, idx, ones, zbuf, acc):
    cid = lax.axis_index("c")
    sid = lax.axis_index("s")
    _zero_vmem(zbuf, 3328)
    one = jnp.full((16,), 1.0, jnp.float32)
    for l in range(CH // 16):
        ones[pl.ds(l * 16, 16)] = one
    pltpu.sync_copy(zbuf, acc.at[pl.ds(sid * 3328, 3328)])
    plsc.subcore_barrier()
    eps = EPAD // 32
    base0 = cid * (EPAD // 2) + sid * eps

    def body(j, carry):
        pltpu.sync_copy(c_hbm.at[pl.ds(base0 + j * CH, CH)], idx)
        pltpu.sync_copy(ones, acc.at[idx], add=True)
        return carry

    lax.fori_loop(0, eps // CH, body, 0)
    plsc.subcore_barrier()
    pltpu.sync_copy(acc.at[pl.ds(sid * 3328, 3328)],
                    out.at[cid, pl.ds(sid * 3328, 3328)])


@functools.partial(
    pl.kernel,
    mesh=_mesh,
    out_type=jax.ShapeDtypeStruct((NQ, NPAD, DQ), jnp.float32),
    scratch_types=[
        pltpu.VMEM((CH,), jnp.int32),
        pltpu.VMEM((CH,), jnp.int32),
        pltpu.VMEM((CH, DQ), jnp.float32),
        pltpu.VMEM((128, DQ), jnp.float32),
        pltpu.VMEM_SHARED((ACCR, DQ), jnp.float32),
        pltpu.SemaphoreType.DMA,
    ],
    compiler_params=_SC_PARAMS,
)
def _prop_kernel(g_hbm, r4_hbm, c_hbm, out, idxg, idxs, rows, zbuf2, acc, sem):
    cid = lax.axis_index("c")
    sid = lax.axis_index("s")
    _zero_vmem2(zbuf2, 128, DQ)
    base0 = sid * EPS
    for ph in range(2):
        q = cid * 2 + ph
        for k in range(3328 // 128):
            pltpu.sync_copy(zbuf2, acc.at[pl.ds(sid * 3328 + k * 128, 128)])
        plsc.subcore_barrier()

        def body(j, carry):
            eb = base0 + j * CH
            pltpu.sync_copy(r4_hbm.at[pl.ds(eb, CH)], idxg)
            pltpu.sync_copy(c_hbm.at[pl.ds(eb, CH)], idxs)
            for l in range(CH // 16):
                idxg[pl.ds(l * 16, 16)] = idxg[pl.ds(l * 16, 16)] + q
            pltpu.async_copy(g_hbm.at[idxg], rows, sem).wait()
            pltpu.sync_copy(rows, acc.at[idxs], add=True)
            return carry

        lax.fori_loop(0, NCH, body, 0)
        plsc.subcore_barrier()
        pltpu.sync_copy(acc.at[pl.ds(sid * 3200, 3200)],
                        out.at[q, pl.ds(sid * 3200, 3200)])
        plsc.subcore_barrier()


@functools.partial(
    pl.kernel,
    mesh=_mesh,
    out_type=jax.ShapeDtypeStruct((2, 640, D), jnp.float32),
    scratch_types=[
        pltpu.VMEM((64,), jnp.int32),
        pltpu.VMEM((64, D), jnp.float32),
        pltpu.VMEM((40, D), jnp.float32),
        pltpu.VMEM_SHARED((640, D), jnp.float32),
    ],
    compiler_params=_SC_PARAMS,
)
def _pool_kernel(a_hbm, b_hbm, out, idx, rows, zbuf2, acc):
    cid = lax.axis_index("c")
    sid = lax.axis_index("s")
    _zero_vmem2(zbuf2, 40, D)
    pltpu.sync_copy(zbuf2, acc.at[pl.ds(sid * 40, 40)])
    plsc.subcore_barrier()
    base0 = cid * (NPAD // 2) + sid * (NPAD // 32)

    def body(j, carry):
        rb = base0 + j * 64
        pltpu.sync_copy(b_hbm.at[pl.ds(rb, 64)], idx)
        pltpu.sync_copy(a_hbm.at[pl.ds(rb, 64)], rows)
        pltpu.sync_copy(rows, acc.at[idx], add=True)
        return carry

    lax.fori_loop(0, (NPAD // 32) // 64, body, 0)
    plsc.subcore_barrier()
    pltpu.sync_copy(acc.at[pl.ds(sid * 40, 40)],
                    out.at[cid, pl.ds(sid * 40, 40)])


def _dis_of(ind_ref):
    ind = ind_ref[0, :] + ind_ref[1, :]
    return lax.rsqrt(ind + 2.0)[:, None]


def _tc_a_body(x_ref, ind_ref, We_ref, be_ref, W1_ref, a0_ref, g1_ref):
    dis = _dis_of(ind_ref)
    a0 = jnp.dot(jnp.log(x_ref[...] + 1.0), We_ref[...],
                 preferred_element_type=jnp.float32) + be_ref[...]
    a0_ref[...] = a0
    g1_ref[...] = dis * jnp.dot(a0, W1_ref[...],
                                preferred_element_type=jnp.float32)


def _tc_layer_body(a_ref, g_ref, s0, s1, s2, s3, ind_ref, W_ref, b_ref,
                   anew_ref, gnew_ref):
    dis = _dis_of(ind_ref)
    s = jnp.concatenate([s0[...], s1[...], s2[...], s3[...]], axis=1)
    anew = a_ref[...] + jnp.maximum(
        dis * (s + 2.0 * g_ref[...]) + b_ref[...], 0.0)
    anew_ref[...] = anew
    gnew_ref[...] = dis * jnp.dot(anew, W_ref[...],
                                  preferred_element_type=jnp.float32)


def _tc_final_body(a_ref, g_ref, s0, s1, s2, s3, ind_ref, b_ref, anew_ref):
    dis = _dis_of(ind_ref)
    s = jnp.concatenate([s0[...], s1[...], s2[...], s3[...]], axis=1)
    anew_ref[...] = a_ref[...] + jnp.maximum(
        dis * (s + 2.0 * g_ref[...]) + b_ref[...], 0.0)


def _row_spec(w=D):
    return pl.BlockSpec((BN, w), lambda i: (i, 0))


def _fix_spec(shape):
    return pl.BlockSpec(shape, lambda i: tuple(0 for _ in shape))


_IND_SPEC = pl.BlockSpec((2, BN), lambda i: (0, i))


def _tc_a(xp, ind2, We, be, W1):
    return pl.pallas_call(
        _tc_a_body,
        grid=(GRID,),
        in_specs=[_row_spec(16), _IND_SPEC, _fix_spec((16, D)),
                  _fix_spec((1, D)), _fix_spec((D, D))],
        out_specs=[_row_spec(), _row_spec()],
        out_shape=[jax.ShapeDtypeStruct((NPAD, D), jnp.float32)] * 2,
    )(xp, ind2, We, be, W1)


def _tc_layer(a, g, s4, ind2, W, b):
    return pl.pallas_call(
        _tc_layer_body,
        grid=(GRID,),
        in_specs=[_row_spec(), _row_spec(),
                  _row_spec(DQ), _row_spec(DQ), _row_spec(DQ), _row_spec(DQ),
                  _IND_SPEC, _fix_spec((D, D)), _fix_spec((1, D))],
        out_specs=[_row_spec(), _row_spec()],
        out_shape=[jax.ShapeDtypeStruct((NPAD, D), jnp.float32)] * 2,
    )(a, g, s4[0], s4[1], s4[2], s4[3], ind2, W, b)


def _tc_final(a, g, s4, ind2, b):
    return pl.pallas_call(
        _tc_final_body,
        grid=(GRID,),
        in_specs=[_row_spec(), _row_spec(),
                  _row_spec(DQ), _row_spec(DQ), _row_spec(DQ), _row_spec(DQ),
                  _IND_SPEC, _fix_spec((1, D))],
        out_specs=[_row_spec()],
        out_shape=[jax.ShapeDtypeStruct((NPAD, D), jnp.float32)],
    )(a, g, s4[0], s4[1], s4[2], s4[3], ind2, b)[0]


def kernel(x, edge_index, batch, W_exp, b_exp, Ws, bs):
    L = Ws.shape[0]
    r = edge_index[0]
    c = edge_index[1]
    xp = jnp.pad(jnp.asarray(x, jnp.float32), ((0, NPAD - N), (0, 5)))
    batchp = jnp.pad(batch.astype(jnp.int32), (0, NPAD - N),
                     constant_values=G)
    j = jnp.arange(EPAD - E, dtype=jnp.int32)
    r4 = jnp.concatenate([r * 4, (j % N) * 4])
    cp = jnp.concatenate([c, NPAD + (j % 8)])
    Wep = jnp.pad(jnp.asarray(W_exp, jnp.float32), ((0, 5), (0, 0)))

    ind2 = _deg_kernel(cp)
    a, g = _tc_a(xp, ind2, Wep, b_exp.reshape(1, D), Ws[0])
    for i in range(1, L + 1):
        s4 = _prop_kernel(g.reshape(NPAD * 4, DQ), r4, cp)
        if i < L:
            a, g = _tc_layer(a, g, s4, ind2, Ws[i], bs[i - 1].reshape(1, D))
        else:
            a = _tc_final(a, g, s4, ind2, bs[L - 1].reshape(1, D))
    parts = _pool_kernel(a, batchp)
    return parts[0, :G] + parts[1, :G]


# speedup vs baseline: 14.4253x; 3.1279x over previous
import functools

import jax
import jax.numpy as jnp
from jax import lax
from jax.experimental import pallas as pl
from jax.experimental.pallas import tpu as pltpu
from jax.experimental.pallas import tpu_sc as plsc

N = 50000
E = 800000
D = 128
G = 512
NPAD = 51200
NQ = 4
DQ = D // NQ
ACCR = 51456
EPS = 51200
EPAD = 16 * EPS
CH = 128
NCH = EPS // CH
SB = 2
SBE = SB * CH
NSUP = EPS // SBE
NT = NSUP // 2
ROWS = EPS // CH
EPADX_ROWS = 16 * ROWS + 2 * (2 * SB)
EPADX = EPADX_ROWS * CH
BN = 1024
GRID = NPAD // BN

_mesh = plsc.VectorSubcoreMesh(core_axis_name="c", subcore_axis_name="s")
_SC_PARAMS = pltpu.CompilerParams(use_tc_tiling_on_sc=False)


def _zero_vmem(ref, nwords):
    zero = jnp.zeros((16,), jnp.float32)

    def body(i, carry):
        ref[pl.ds(i * 16, 16)] = zero
        return carry

    lax.fori_loop(0, nwords // 16, body, 0)


def _zero_vmem2(ref, nrows, ncols):
    zero = jnp.zeros((16,), jnp.float32)

    def body(i, carry):
        for l in range(ncols // 16):
            ref[i, pl.ds(l * 16, 16)] = zero
        return carry

    lax.fori_loop(0, nrows, body, 0)


@functools.partial(
    pl.kernel,
    mesh=_mesh,
    out_type=jax.ShapeDtypeStruct((2, ACCR), jnp.float32),
    scratch_types=[
        pltpu.VMEM((CH,), jnp.int32),
        pltpu.VMEM((CH,), jnp.float32),
        pltpu.VMEM((3216,), jnp.float32),
        pltpu.VMEM_SHARED((ACCR,), jnp.float32),
    ],
    compiler_params=_SC_PARAMS,
)
def _deg_kernel(c_hbm, out, idx, ones, zbuf, acc):
    cid = lax.axis_index("c")
    sid = lax.axis_index("s")
    _zero_vmem(zbuf, 3216)
    one = jnp.full((16,), 1.0, jnp.float32)
    for l in range(CH // 16):
        ones[pl.ds(l * 16, 16)] = one
    pltpu.sync_copy(zbuf, acc.at[pl.ds(sid * 3216, 3216)])
    plsc.subcore_barrier()
    eps = EPAD // 32
    base0 = cid * (EPAD // 2) + sid * eps

    def body(j, carry):
        pltpu.sync_copy(c_hbm.at[pl.ds(base0 + j * CH, CH)], idx)
        pltpu.sync_copy(ones, acc.at[idx], add=True)
        return carry

    lax.fori_loop(0, eps // CH, body, 0)
    plsc.subcore_barrier()
    pltpu.sync_copy(acc.at[pl.ds(sid * 3216, 3216)],
                    out.at[cid, pl.ds(sid * 3216, 3216)])


@functools.partial(
    pl.kernel,
    mesh=_mesh,
    out_type=jax.ShapeDtypeStruct((NQ, NPAD, DQ), jnp.float32),
    scratch_types=[
        pltpu.VMEM((2 * SB, CH), jnp.int32),
        pltpu.VMEM((2 * SB, CH), jnp.int32),
        pltpu.VMEM((2 * SB, CH), jnp.int32),
        pltpu.VMEM((2 * SB, CH), jnp.int32),
        pltpu.VMEM((SBE, DQ), jnp.float32),
        pltpu.VMEM((SBE, DQ), jnp.float32),
        pltpu.VMEM((64, DQ), jnp.float32),
        pltpu.VMEM_SHARED((ACCR, DQ), jnp.float32),
        pltpu.SemaphoreType.DMA,
        pltpu.SemaphoreType.DMA,
        pltpu.SemaphoreType.DMA,
        pltpu.SemaphoreType.DMA,
        pltpu.SemaphoreType.DMA,
    ],
    compiler_params=_SC_PARAMS,
)
def _prop_kernel(g_hbm, r4q_hbm, c2_hbm, out, idxg0, idxg1, idxs0, idxs1,
                 rowsA, rowsB, zbuf2, acc, semA, semB, semSA, semSB, semI):
    cid = lax.axis_index("c")
    sid = lax.axis_index("s")
    _zero_vmem2(zbuf2, 64, DQ)
    br = sid * ROWS

    def fire_gathers(idxg, lo, rows, sem):
        for i in range(SB):
            pltpu.async_copy(g_hbm.at[idxg.at[lo + i]],
                             rows.at[pl.ds(i * CH, CH)], sem)

    def fire_scatters(idxs, lo, rows, sem):
        for i in range(SB):
            pltpu.async_copy(rows.at[pl.ds(i * CH, CH)],
                             acc.at[idxs.at[lo + i]], sem, add=True)

    def drain_gather(rows, sem):
        pltpu.make_async_copy(g_hbm.at[pl.ds(0, SBE)], rows, sem).wait()

    def drain_scatter(rows, sem):
        pltpu.make_async_copy(rows, acc.at[pl.ds(0, SBE)], sem).wait()

    def stage_idx(q, row0, idxg, idxs, sync=False):
        if sync:
            pltpu.sync_copy(r4q_hbm.at[q, pl.ds(row0, 2 * SB)], idxg)
            pltpu.sync_copy(c2_hbm.at[pl.ds(row0, 2 * SB)], idxs)
        else:
            pltpu.async_copy(r4q_hbm.at[q, pl.ds(row0, 2 * SB)], idxg, semI)
            pltpu.async_copy(c2_hbm.at[pl.ds(row0, 2 * SB)], idxs, semI)

    def drain_idx(q, idxg, idxs):
        pltpu.make_async_copy(r4q_hbm.at[q, pl.ds(0, 2 * SB)], idxg,
                              semI).wait()
        pltpu.make_async_copy(c2_hbm.at[pl.ds(0, 2 * SB)], idxs, semI).wait()

    for ph in range(2):
        q = cid * 2 + ph
        for k in range(50):
            pltpu.sync_copy(zbuf2, acc.at[pl.ds(sid * 3216 + k * 64, 64)])
        pltpu.sync_copy(zbuf2.at[pl.ds(0, 16)],
                        acc.at[pl.ds(sid * 3216 + 3200, 16)])
        plsc.subcore_barrier()

        stage_idx(q, br, idxg0, idxs0, sync=True)
        fire_gathers(idxg0, 0, rowsA, semA)

        def body(t, carry):
            def halfs(idxgX, idxsX, idxgY, idxsY):
                @pl.when(t > 0)
                def _():
                    drain_scatter(rowsB, semSB)
                stage_idx(q, br + (2 * t + 2) * SB, idxgY, idxsY)
                fire_gathers(idxgX, SB, rowsB, semB)
                drain_gather(rowsA, semA)
                fire_scatters(idxsX, 0, rowsA, semSA)
                drain_scatter(rowsA, semSA)
                drain_idx(q, idxgY, idxsY)
                fire_gathers(idxgY, 0, rowsA, semA)
                drain_gather(rowsB, semB)
                fire_scatters(idxsX, SB, rowsB, semSB)

            lax.cond(t % 2 == 0,
                     lambda: halfs(idxg0, idxs0, idxg1, idxs1),
                     lambda: halfs(idxg1, idxs1, idxg0, idxs0))
            return carry

        lax.fori_loop(0, NT, body, 0)
        drain_scatter(rowsB, semSB)
        drain_gather(rowsA, semA)
        plsc.subcore_barrier()
        pltpu.sync_copy(acc.at[pl.ds(sid * 3200, 3200)],
                        out.at[q, pl.ds(sid * 3200, 3200)])
        plsc.subcore_barrier()


@functools.partial(
    pl.kernel,
    mesh=_mesh,
    out_type=jax.ShapeDtypeStruct((2, 640, D), jnp.float32),
    scratch_types=[
        pltpu.VMEM((64,), jnp.int32),
        pltpu.VMEM((64, D), jnp.float32),
        pltpu.VMEM((40, D), jnp.float32),
        pltpu.VMEM_SHARED((640, D), jnp.float32),
    ],
    compiler_params=_SC_PARAMS,
)
def _pool_kernel(a_hbm, b_hbm, out, idx, rows, zbuf2, acc):
    cid = lax.axis_index("c")
    sid = lax.axis_index("s")
    _zero_vmem2(zbuf2, 40, D)
    pltpu.sync_copy(zbuf2, acc.at[pl.ds(sid * 40, 40)])
    plsc.subcore_barrier()
    base0 = cid * (NPAD // 2) + sid * (NPAD // 32)

    def body(j, carry):
        rb = base0 + j * 64
        pltpu.sync_copy(b_hbm.at[pl.ds(rb, 64)], idx)
        pltpu.sync_copy(a_hbm.at[pl.ds(rb, 64)], rows)
        pltpu.sync_copy(rows, acc.at[idx], add=True)
        return carry

    lax.fori_loop(0, (NPAD // 32) // 64, body, 0)
    plsc.subcore_barrier()
    pltpu.sync_copy(acc.at[pl.ds(sid * 40, 40)],
                    out.at[cid, pl.ds(sid * 40, 40)])


def _dis_of(ind_ref):
    ind = ind_ref[0, :] + ind_ref[1, :]
    return lax.rsqrt(ind + 2.0)[:, None]


def _tc_a_body(x_ref, ind_ref, We_ref, be_ref, W1_ref, a0_ref, g1_ref):
    dis = _dis_of(ind_ref)
    a0 = jnp.dot(jnp.log(x_ref[...] + 1.0), We_ref[...],
                 preferred_element_type=jnp.float32) + be_ref[...]
    a0_ref[...] = a0
    g1_ref[...] = dis * jnp.dot(a0, W1_ref[...],
                                preferred_element_type=jnp.float32)


def _tc_layer_body(a_ref, g_ref, s0, s1, s2, s3, ind_ref, W_ref, b_ref,
                   anew_ref, gnew_ref):
    dis = _dis_of(ind_ref)
    s = jnp.concatenate([s0[...], s1[...], s2[...], s3[...]], axis=1)
    anew = a_ref[...] + jnp.maximum(
        dis * (s + 2.0 * g_ref[...]) + b_ref[...], 0.0)
    anew_ref[...] = anew
    gnew_ref[...] = dis * jnp.dot(anew, W_ref[...],
                                  preferred_element_type=jnp.float32)


def _tc_final_body(a_ref, g_ref, s0, s1, s2, s3, ind_ref, b_ref, anew_ref):
    dis = _dis_of(ind_ref)
    s = jnp.concatenate([s0[...], s1[...], s2[...], s3[...]], axis=1)
    anew_ref[...] = a_ref[...] + jnp.maximum(
        dis * (s + 2.0 * g_ref[...]) + b_ref[...], 0.0)


def _row_spec(w=D):
    return pl.BlockSpec((BN, w), lambda i: (i, 0))


def _fix_spec(shape):
    return pl.BlockSpec(shape, lambda i: tuple(0 for _ in shape))


_IND_SPEC = pl.BlockSpec((2, BN), lambda i: (0, i))


def _tc_a(xp, ind2, We, be, W1):
    return pl.pallas_call(
        _tc_a_body,
        grid=(GRID,),
        in_specs=[_row_spec(16), _IND_SPEC, _fix_spec((16, D)),
                  _fix_spec((1, D)), _fix_spec((D, D))],
        out_specs=[_row_spec(), _row_spec()],
        out_shape=[jax.ShapeDtypeStruct((NPAD, D), jnp.float32)] * 2,
    )(xp, ind2, We, be, W1)


def _tc_layer(a, g, s4, ind2, W, b):
    return pl.pallas_call(
        _tc_layer_body,
        grid=(GRID,),
        in_specs=[_row_spec(), _row_spec(),
                  _row_spec(DQ), _row_spec(DQ), _row_spec(DQ), _row_spec(DQ),
                  _IND_SPEC, _fix_spec((D, D)), _fix_spec((1, D))],
        out_specs=[_row_spec(), _row_spec()],
        out_shape=[jax.ShapeDtypeStruct((NPAD, D), jnp.float32)] * 2,
    )(a, g, s4[0], s4[1], s4[2], s4[3], ind2, W, b)


def _tc_final(a, g, s4, ind2, b):
    return pl.pallas_call(
        _tc_final_body,
        grid=(GRID,),
        in_specs=[_row_spec(), _row_spec(),
                  _row_spec(DQ), _row_spec(DQ), _row_spec(DQ), _row_spec(DQ),
                  _IND_SPEC, _fix_spec((1, D))],
        out_specs=[_row_spec()],
        out_shape=[jax.ShapeDtypeStruct((NPAD, D), jnp.float32)],
    )(a, g, s4[0], s4[1], s4[2], s4[3], ind2, b)[0]


def kernel(x, edge_index, batch, W_exp, b_exp, Ws, bs):
    L = Ws.shape[0]
    r = edge_index[0]
    c = edge_index[1]
    xp = jnp.pad(jnp.asarray(x, jnp.float32), ((0, NPAD - N), (0, 5)))
    batchp = jnp.pad(batch.astype(jnp.int32), (0, NPAD - N),
                     constant_values=G)
    j = jnp.arange(EPADX - E, dtype=jnp.int32)
    rfull = jnp.concatenate([r, j % 4096])
    cflat = jnp.concatenate([c, NPAD + (j % 8)])
    r4q3 = ((rfull * 4)[None, :]
            + jnp.arange(4, dtype=jnp.int32)[:, None]).reshape(
                4, EPADX_ROWS, CH)
    c2 = cflat.reshape(EPADX_ROWS, CH)
    Wep = jnp.pad(jnp.asarray(W_exp, jnp.float32), ((0, 5), (0, 0)))

    ind2 = _deg_kernel(cflat)
    a, g = _tc_a(xp, ind2, Wep, b_exp.reshape(1, D), Ws[0])
    for i in range(1, L + 1):
        s4 = _prop_kernel(g.reshape(NPAD * 4, DQ), r4q3, c2)
        if i < L:
            a, g = _tc_layer(a, g, s4, ind2, Ws[i], bs[i - 1].reshape(1, D))
        else:
            a = _tc_final(a, g, s4, ind2, bs[L - 1].reshape(1, D))
    parts = _pool_kernel(a, batchp)
    return parts[0, :G] + parts[1, :G]


# speedup vs baseline: 15.4225x; 1.0691x over previous
import functools

import jax
import jax.numpy as jnp
from jax import lax
from jax.experimental import pallas as pl
from jax.experimental.pallas import tpu as pltpu
from jax.experimental.pallas import tpu_sc as plsc

N = 50000
E = 800000
D = 128
G = 512
NPAD = 51200
NQ = 4
DQ = D // NQ
ACCR = 51456
EPS = 51200
EPAD = 16 * EPS
CH = 128
NCH = EPS // CH
SB = 2
SBE = SB * CH
NSUP = EPS // SBE
NT = NSUP // 2
ROWS = EPS // CH
EPADX_ROWS = 16 * ROWS + 2 * (2 * SB)
EPADX = EPADX_ROWS * CH
BN = 1024
GRID = NPAD // BN

_mesh = plsc.VectorSubcoreMesh(core_axis_name="c", subcore_axis_name="s")
_SC_PARAMS = pltpu.CompilerParams(use_tc_tiling_on_sc=False)


def _zero_vmem(ref, nwords):
    zero = jnp.zeros((16,), jnp.float32)

    def body(i, carry):
        ref[pl.ds(i * 16, 16)] = zero
        return carry

    lax.fori_loop(0, nwords // 16, body, 0)


def _zero_vmem2(ref, nrows, ncols):
    zero = jnp.zeros((16,), jnp.float32)

    def body(i, carry):
        for l in range(ncols // 16):
            ref[i, pl.ds(l * 16, 16)] = zero
        return carry

    lax.fori_loop(0, nrows, body, 0)


@functools.partial(
    pl.kernel,
    mesh=_mesh,
    out_type=jax.ShapeDtypeStruct((2, ACCR), jnp.float32),
    scratch_types=[
        pltpu.VMEM((CH,), jnp.int32),
        pltpu.VMEM((CH,), jnp.float32),
        pltpu.VMEM((3216,), jnp.float32),
        pltpu.VMEM_SHARED((ACCR,), jnp.float32),
    ],
    compiler_params=_SC_PARAMS,
)
def _deg_kernel(c_hbm, out, idx, ones, zbuf, acc):
    cid = lax.axis_index("c")
    sid = lax.axis_index("s")
    _zero_vmem(zbuf, 3216)
    one = jnp.full((16,), 1.0, jnp.float32)
    for l in range(CH // 16):
        ones[pl.ds(l * 16, 16)] = one
    pltpu.sync_copy(zbuf, acc.at[pl.ds(sid * 3216, 3216)])
    plsc.subcore_barrier()
    eps = EPAD // 32
    base0 = cid * (EPAD // 2) + sid * eps

    def body(j, carry):
        pltpu.sync_copy(c_hbm.at[pl.ds(base0 + j * CH, CH)], idx)
        pltpu.sync_copy(ones, acc.at[idx], add=True)
        return carry

    lax.fori_loop(0, eps // CH, body, 0)
    plsc.subcore_barrier()
    pltpu.sync_copy(acc.at[pl.ds(sid * 3216, 3216)],
                    out.at[cid, pl.ds(sid * 3216, 3216)])


NIB = 8
NBLKP = ROWS // NIB


@functools.partial(
    pl.kernel,
    mesh=_mesh,
    out_type=jax.ShapeDtypeStruct((NQ, NPAD, DQ), jnp.float32),
    scratch_types=[
        pltpu.VMEM((NIB, CH), jnp.int32),
        pltpu.VMEM((NIB, CH), jnp.int32),
        pltpu.VMEM((NIB, CH), jnp.int32),
        pltpu.VMEM((NIB, CH), jnp.int32),
        pltpu.VMEM((NIB, CH), jnp.int32),
        pltpu.VMEM((NIB, CH), jnp.int32),
        pltpu.VMEM((NIB, CH), jnp.int32),
        pltpu.VMEM((NIB, CH), jnp.int32),
        pltpu.VMEM((CH, DQ), jnp.float32),
        pltpu.VMEM((CH, DQ), jnp.float32),
        pltpu.VMEM((CH, DQ), jnp.float32),
        pltpu.VMEM((CH, DQ), jnp.float32),
        pltpu.VMEM((64, DQ), jnp.float32),
        pltpu.VMEM_SHARED((ACCR, DQ), jnp.float32),
        pltpu.SemaphoreType.DMA,
        pltpu.SemaphoreType.DMA,
        pltpu.SemaphoreType.DMA,
        pltpu.SemaphoreType.DMA,
        pltpu.SemaphoreType.DMA,
        pltpu.SemaphoreType.DMA,
        pltpu.SemaphoreType.DMA,
        pltpu.SemaphoreType.DMA,
        pltpu.SemaphoreType.DMA,
    ],
    compiler_params=_SC_PARAMS,
)
def _prop_kernel(g_hbm, r4q_hbm, c2_hbm, out,
                 ig0, ig1, ig2, ig3, is0, is1, is2, is3,
                 rb0, rb1, rb2, rb3, zbuf2, acc,
                 sg0, sg1, sg2, sg3, ss0, ss1, ss2, ss3, semI):
    cid = lax.axis_index("c")
    sid = lax.axis_index("s")
    _zero_vmem2(zbuf2, 64, DQ)
    br = sid * ROWS
    IG = [ig0, ig1, ig2, ig3]
    IS = [is0, is1, is2, is3]
    RB = [rb0, rb1, rb2, rb3]
    SG = [sg0, sg1, sg2, sg3]
    SS = [ss0, ss1, ss2, ss3]

    def wait_scat(b):
        pltpu.make_async_copy(RB[b], acc.at[pl.ds(0, CH)], SS[b]).wait()

    def wait_gath(b):
        pltpu.make_async_copy(g_hbm.at[pl.ds(0, CH)], RB[b], SG[b]).wait()

    def fire_gather(s, j, b):
        pltpu.async_copy(g_hbm.at[IG[s].at[j]], RB[b], SG[b])

    def fire_scatter(s, j, b):
        pltpu.async_copy(RB[b], acc.at[IS[s].at[j]], SS[b], add=True)

    def stage(q, blk, s, sync=False):
        if sync:
            pltpu.sync_copy(r4q_hbm.at[q, pl.ds(br + blk * NIB, NIB)], IG[s])
            pltpu.sync_copy(c2_hbm.at[pl.ds(br + blk * NIB, NIB)], IS[s])
        else:
            pltpu.async_copy(r4q_hbm.at[q, pl.ds(br + blk * NIB, NIB)],
                             IG[s], semI)
            pltpu.async_copy(c2_hbm.at[pl.ds(br + blk * NIB, NIB)],
                             IS[s], semI)

    def drain_stage(q, s):
        pltpu.make_async_copy(r4q_hbm.at[q, pl.ds(0, NIB)], IG[s],
                              semI).wait()
        pltpu.make_async_copy(c2_hbm.at[pl.ds(0, NIB)], IS[s], semI).wait()

    for ph in range(2):
        q = cid * 2 + ph
        for k in range(50):
            pltpu.sync_copy(zbuf2, acc.at[pl.ds(sid * 3216 + k * 64, 64)])
        pltpu.sync_copy(zbuf2.at[pl.ds(0, 16)],
                        acc.at[pl.ds(sid * 3216 + 3200, 16)])
        plsc.subcore_barrier()

        stage(q, 0, 0, sync=True)
        stage(q, 1, 1)
        stage(q, 2, 2)

        def block_body(blk, s):
            sp = (s + 3) % 4
            @pl.when(blk >= 1)
            def _():
                drain_stage(q, s)
            for j in range(NIB):
                if j == 4:
                    @pl.when(blk + 3 <= NBLKP - 1)
                    def _():
                        stage(q, blk + 3, sp)
                b = j % 4
                if j >= 4:
                    wait_scat(b)
                else:
                    @pl.when(blk >= 1)
                    def _(b=b):
                        wait_scat(b)
                fire_gather(s, j, b)
                bd = (j + 2) % 4
                if j >= 2:
                    wait_gath(bd)
                    fire_scatter(s, j - 2, bd)
                else:
                    @pl.when(blk >= 1)
                    def _(j=j, bd=bd):
                        wait_gath(bd)
                        fire_scatter(sp, j + 6, bd)

        def body(blk, carry):
            m4 = blk % 4
            lax.cond(
                m4 % 2 == 0,
                lambda: lax.cond(m4 == 0,
                                 lambda: block_body(blk, 0),
                                 lambda: block_body(blk, 2)),
                lambda: lax.cond(m4 == 1,
                                 lambda: block_body(blk, 1),
                                 lambda: block_body(blk, 3)))
            return carry

        lax.fori_loop(0, NBLKP, body, 0)
        s_last = (NBLKP - 1) % 4
        for dj in (6, 7):
            bd = dj % 4
            wait_gath(bd)
            fire_scatter(s_last, dj, bd)
        for b in range(4):
            wait_scat(b)
        plsc.subcore_barrier()
        pltpu.sync_copy(acc.at[pl.ds(sid * 3200, 3200)],
                        out.at[q, pl.ds(sid * 3200, 3200)])
        plsc.subcore_barrier()


@functools.partial(
    pl.kernel,
    mesh=_mesh,
    out_type=jax.ShapeDtypeStruct((2, 640, D), jnp.float32),
    scratch_types=[
        pltpu.VMEM((64,), jnp.int32),
        pltpu.VMEM((64, D), jnp.float32),
        pltpu.VMEM((40, D), jnp.float32),
        pltpu.VMEM_SHARED((640, D), jnp.float32),
    ],
    compiler_params=_SC_PARAMS,
)
def _pool_kernel(a_hbm, b_hbm, out, idx, rows, zbuf2, acc):
    cid = lax.axis_index("c")
    sid = lax.axis_index("s")
    _zero_vmem2(zbuf2, 40, D)
    pltpu.sync_copy(zbuf2, acc.at[pl.ds(sid * 40, 40)])
    plsc.subcore_barrier()
    base0 = cid * (NPAD // 2) + sid * (NPAD // 32)

    def body(j, carry):
        rb = base0 + j * 64
        pltpu.sync_copy(b_hbm.at[pl.ds(rb, 64)], idx)
        pltpu.sync_copy(a_hbm.at[pl.ds(rb, 64)], rows)
        pltpu.sync_copy(rows, acc.at[idx], add=True)
        return carry

    lax.fori_loop(0, (NPAD // 32) // 64, body, 0)
    plsc.subcore_barrier()
    pltpu.sync_copy(acc.at[pl.ds(sid * 40, 40)],
                    out.at[cid, pl.ds(sid * 40, 40)])


def _dis_of(ind_ref):
    ind = ind_ref[0, :] + ind_ref[1, :]
    return lax.rsqrt(ind + 2.0)[:, None]


def _tc_a_body(x_ref, ind_ref, We_ref, be_ref, W1_ref, a0_ref, g1_ref):
    dis = _dis_of(ind_ref)
    a0 = jnp.dot(jnp.log(x_ref[...] + 1.0), We_ref[...],
                 preferred_element_type=jnp.float32) + be_ref[...]
    a0_ref[...] = a0
    g1_ref[...] = dis * jnp.dot(a0, W1_ref[...],
                                preferred_element_type=jnp.float32)


def _tc_layer_body(a_ref, g_ref, s0, s1, s2, s3, ind_ref, W_ref, b_ref,
                   anew_ref, gnew_ref):
    dis = _dis_of(ind_ref)
    s = jnp.concatenate([s0[...], s1[...], s2[...], s3[...]], axis=1)
    anew = a_ref[...] + jnp.maximum(
        dis * (s + 2.0 * g_ref[...]) + b_ref[...], 0.0)
    anew_ref[...] = anew
    gnew_ref[...] = dis * jnp.dot(anew, W_ref[...],
                                  preferred_element_type=jnp.float32)


def _tc_final_body(a_ref, g_ref, s0, s1, s2, s3, ind_ref, b_ref, anew_ref):
    dis = _dis_of(ind_ref)
    s = jnp.concatenate([s0[...], s1[...], s2[...], s3[...]], axis=1)
    anew_ref[...] = a_ref[...] + jnp.maximum(
        dis * (s + 2.0 * g_ref[...]) + b_ref[...], 0.0)


def _row_spec(w=D):
    return pl.BlockSpec((BN, w), lambda i: (i, 0))


def _fix_spec(shape):
    return pl.BlockSpec(shape, lambda i: tuple(0 for _ in shape))


_IND_SPEC = pl.BlockSpec((2, BN), lambda i: (0, i))


def _tc_a(xp, ind2, We, be, W1):
    return pl.pallas_call(
        _tc_a_body,
        grid=(GRID,),
        in_specs=[_row_spec(16), _IND_SPEC, _fix_spec((16, D)),
                  _fix_spec((1, D)), _fix_spec((D, D))],
        out_specs=[_row_spec(), _row_spec()],
        out_shape=[jax.ShapeDtypeStruct((NPAD, D), jnp.float32)] * 2,
    )(xp, ind2, We, be, W1)


def _tc_layer(a, g, s4, ind2, W, b):
    return pl.pallas_call(
        _tc_layer_body,
        grid=(GRID,),
        in_specs=[_row_spec(), _row_spec(),
                  _row_spec(DQ), _row_spec(DQ), _row_spec(DQ), _row_spec(DQ),
                  _IND_SPEC, _fix_spec((D, D)), _fix_spec((1, D))],
        out_specs=[_row_spec(), _row_spec()],
        out_shape=[jax.ShapeDtypeStruct((NPAD, D), jnp.float32)] * 2,
    )(a, g, s4[0], s4[1], s4[2], s4[3], ind2, W, b)


def _tc_final(a, g, s4, ind2, b):
    return pl.pallas_call(
        _tc_final_body,
        grid=(GRID,),
        in_specs=[_row_spec(), _row_spec(),
                  _row_spec(DQ), _row_spec(DQ), _row_spec(DQ), _row_spec(DQ),
                  _IND_SPEC, _fix_spec((1, D))],
        out_specs=[_row_spec()],
        out_shape=[jax.ShapeDtypeStruct((NPAD, D), jnp.float32)],
    )(a, g, s4[0], s4[1], s4[2], s4[3], ind2, b)[0]


def kernel(x, edge_index, batch, W_exp, b_exp, Ws, bs):
    L = Ws.shape[0]
    r = edge_index[0]
    c = edge_index[1]
    xp = jnp.pad(jnp.asarray(x, jnp.float32), ((0, NPAD - N), (0, 5)))
    batchp = jnp.pad(batch.astype(jnp.int32), (0, NPAD - N),
                     constant_values=G)
    j = jnp.arange(EPADX - E, dtype=jnp.int32)
    rfull = jnp.concatenate([r, j % 4096])
    cflat = jnp.concatenate([c, NPAD + (j % 8)])
    r4q3 = ((rfull * 4)[None, :]
            + jnp.arange(4, dtype=jnp.int32)[:, None]).reshape(
                4, EPADX_ROWS, CH)
    c2 = cflat.reshape(EPADX_ROWS, CH)
    Wep = jnp.pad(jnp.asarray(W_exp, jnp.float32), ((0, 5), (0, 0)))

    ind2 = _deg_kernel(cflat)
    a, g = _tc_a(xp, ind2, Wep, b_exp.reshape(1, D), Ws[0])
    for i in range(1, L + 1):
        s4 = _prop_kernel(g.reshape(NPAD * 4, DQ), r4q3, c2)
        if i < L:
            a, g = _tc_layer(a, g, s4, ind2, Ws[i], bs[i - 1].reshape(1, D))
        else:
            a = _tc_final(a, g, s4, ind2, bs[L - 1].reshape(1, D))
    parts = _pool_kernel(a, batchp)
    return parts[0, :G] + parts[1, :G]


# speedup vs baseline: 16.0200x; 1.0387x over previous
import functools

import jax
import jax.numpy as jnp
from jax import lax
from jax.experimental import pallas as pl
from jax.experimental.pallas import tpu as pltpu
from jax.experimental.pallas import tpu_sc as plsc

N = 50000
E = 800000
D = 128
G = 512
NPAD = 51200
NQ = 4
DQ = D // NQ
ACCR = 51456
EPS = 51200
EPAD = 16 * EPS
CH = 128
NCH = EPS // CH
SB = 2
SBE = SB * CH
NSUP = EPS // SBE
NT = NSUP // 2
ROWS = EPS // CH
EPADX_ROWS = 16 * ROWS + 2 * (2 * SB)
EPADX = EPADX_ROWS * CH
BN = 1024
GRID = NPAD // BN

_mesh = plsc.VectorSubcoreMesh(core_axis_name="c", subcore_axis_name="s")
_SC_PARAMS = pltpu.CompilerParams(use_tc_tiling_on_sc=False)


def _zero_vmem(ref, nwords):
    zero = jnp.zeros((16,), jnp.float32)

    def body(i, carry):
        ref[pl.ds(i * 16, 16)] = zero
        return carry

    lax.fori_loop(0, nwords // 16, body, 0)


def _zero_vmem2(ref, nrows, ncols):
    zero = jnp.zeros((16,), jnp.float32)

    def body(i, carry):
        for l in range(ncols // 16):
            ref[i, pl.ds(l * 16, 16)] = zero
        return carry

    lax.fori_loop(0, nrows, body, 0)


DROWS = 200


@functools.partial(
    pl.kernel,
    mesh=_mesh,
    out_type=jax.ShapeDtypeStruct((2, ACCR), jnp.float32),
    scratch_types=[
        pltpu.VMEM((DROWS, CH), jnp.int32),
        pltpu.VMEM((CH,), jnp.float32),
        pltpu.VMEM((3216,), jnp.float32),
        pltpu.VMEM_SHARED((ACCR,), jnp.float32),
        pltpu.SemaphoreType.DMA,
    ],
    compiler_params=_SC_PARAMS,
)
def _deg_kernel(c2_hbm, out, idx, ones, zbuf, acc, semS):
    cid = lax.axis_index("c")
    sid = lax.axis_index("s")
    _zero_vmem(zbuf, 3216)
    one = jnp.full((16,), 1.0, jnp.float32)
    for l in range(CH // 16):
        ones[pl.ds(l * 16, 16)] = one
    pltpu.sync_copy(zbuf, acc.at[pl.ds(sid * 3216, 3216)])
    plsc.subcore_barrier()
    pltpu.sync_copy(c2_hbm.at[pl.ds(cid * (DROWS * 16) + sid * DROWS, DROWS)],
                    idx)

    def body(j, carry):
        pltpu.async_copy(ones, acc.at[idx.at[j]], semS, add=True)

        @pl.when(j >= 8)
        def _():
            pltpu.make_async_copy(ones, acc.at[pl.ds(0, CH)], semS).wait()

        return carry

    lax.fori_loop(0, DROWS, body, 0)
    for _ in range(8):
        pltpu.make_async_copy(ones, acc.at[pl.ds(0, CH)], semS).wait()
    plsc.subcore_barrier()
    pltpu.sync_copy(acc.at[pl.ds(sid * 3216, 3216)],
                    out.at[cid, pl.ds(sid * 3216, 3216)])


NIB = 8
NBLKP = ROWS // NIB


@functools.partial(
    pl.kernel,
    mesh=_mesh,
    out_type=jax.ShapeDtypeStruct((NQ, NPAD, DQ), jnp.float32),
    scratch_types=[
        pltpu.VMEM((NIB, CH), jnp.int32),
        pltpu.VMEM((NIB, CH), jnp.int32),
        pltpu.VMEM((NIB, CH), jnp.int32),
        pltpu.VMEM((NIB, CH), jnp.int32),
        pltpu.VMEM((NIB, CH), jnp.int32),
        pltpu.VMEM((NIB, CH), jnp.int32),
        pltpu.VMEM((NIB, CH), jnp.int32),
        pltpu.VMEM((NIB, CH), jnp.int32),
        pltpu.VMEM((CH, DQ), jnp.float32),
        pltpu.VMEM((CH, DQ), jnp.float32),
        pltpu.VMEM((CH, DQ), jnp.float32),
        pltpu.VMEM((CH, DQ), jnp.float32),
        pltpu.VMEM((64, DQ), jnp.float32),
        pltpu.VMEM_SHARED((ACCR, DQ), jnp.float32),
        pltpu.SemaphoreType.DMA,
        pltpu.SemaphoreType.DMA,
        pltpu.SemaphoreType.DMA,
        pltpu.SemaphoreType.DMA,
        pltpu.SemaphoreType.DMA,
        pltpu.SemaphoreType.DMA,
        pltpu.SemaphoreType.DMA,
        pltpu.SemaphoreType.DMA,
        pltpu.SemaphoreType.DMA,
    ],
    compiler_params=_SC_PARAMS,
)
def _prop_kernel(g_hbm, r4q_hbm, c2_hbm, out,
                 ig0, ig1, ig2, ig3, is0, is1, is2, is3,
                 rb0, rb1, rb2, rb3, zbuf2, acc,
                 sg0, sg1, sg2, sg3, ss0, ss1, ss2, ss3, semI):
    cid = lax.axis_index("c")
    sid = lax.axis_index("s")
    _zero_vmem2(zbuf2, 64, DQ)
    br = sid * ROWS
    IG = [ig0, ig1, ig2, ig3]
    IS = [is0, is1, is2, is3]
    RB = [rb0, rb1, rb2, rb3]
    SG = [sg0, sg1, sg2, sg3]
    SS = [ss0, ss1, ss2, ss3]

    def wait_scat(b):
        pltpu.make_async_copy(RB[b], acc.at[pl.ds(0, CH)], SS[b]).wait()

    def wait_gath(b):
        pltpu.make_async_copy(g_hbm.at[pl.ds(0, CH)], RB[b], SG[b]).wait()

    def fire_gather(s, j, b):
        pltpu.async_copy(g_hbm.at[IG[s].at[j]], RB[b], SG[b])

    def fire_scatter(s, j, b):
        pltpu.async_copy(RB[b], acc.at[IS[s].at[j]], SS[b], add=True)

    def stage(q, blk, s, sync=False):
        if sync:
            pltpu.sync_copy(r4q_hbm.at[q, pl.ds(br + blk * NIB, NIB)], IG[s])
            pltpu.sync_copy(c2_hbm.at[pl.ds(br + blk * NIB, NIB)], IS[s])
        else:
            pltpu.async_copy(r4q_hbm.at[q, pl.ds(br + blk * NIB, NIB)],
                             IG[s], semI)
            pltpu.async_copy(c2_hbm.at[pl.ds(br + blk * NIB, NIB)],
                             IS[s], semI)

    def drain_stage(q, s):
        pltpu.make_async_copy(r4q_hbm.at[q, pl.ds(0, NIB)], IG[s],
                              semI).wait()
        pltpu.make_async_copy(c2_hbm.at[pl.ds(0, NIB)], IS[s], semI).wait()

    for ph in range(2):
        q = cid * 2 + ph
        for k in range(50):
            pltpu.sync_copy(zbuf2, acc.at[pl.ds(sid * 3216 + k * 64, 64)])
        pltpu.sync_copy(zbuf2.at[pl.ds(0, 16)],
                        acc.at[pl.ds(sid * 3216 + 3200, 16)])
        plsc.subcore_barrier()

        stage(q, 0, 0, sync=True)
        stage(q, 1, 1)
        stage(q, 2, 2)

        def block_body(blk, s):
            sp = (s + 3) % 4
            @pl.when(blk >= 1)
            def _():
                drain_stage(q, s)
            for j in range(NIB):
                if j == 4:
                    @pl.when(blk + 3 <= NBLKP - 1)
                    def _():
                        stage(q, blk + 3, sp)
                b = j % 4
                if j >= 4:
                    wait_scat(b)
                else:
                    @pl.when(blk >= 1)
                    def _(b=b):
                        wait_scat(b)
                fire_gather(s, j, b)
                bd = (j + 2) % 4
                if j >= 2:
                    wait_gath(bd)
                    fire_scatter(s, j - 2, bd)
                else:
                    @pl.when(blk >= 1)
                    def _(j=j, bd=bd):
                        wait_gath(bd)
                        fire_scatter(sp, j + 6, bd)

        def body(blk, carry):
            m4 = blk % 4
            lax.cond(
                m4 % 2 == 0,
                lambda: lax.cond(m4 == 0,
                                 lambda: block_body(blk, 0),
                                 lambda: block_body(blk, 2)),
                lambda: lax.cond(m4 == 1,
                                 lambda: block_body(blk, 1),
                                 lambda: block_body(blk, 3)))
            return carry

        lax.fori_loop(0, NBLKP, body, 0)
        s_last = (NBLKP - 1) % 4
        for dj in (6, 7):
            bd = dj % 4
            wait_gath(bd)
            fire_scatter(s_last, dj, bd)
        for b in range(4):
            wait_scat(b)
        plsc.subcore_barrier()
        pltpu.sync_copy(acc.at[pl.ds(sid * 3200, 3200)],
                        out.at[q, pl.ds(sid * 3200, 3200)])
        plsc.subcore_barrier()


PJ = (NPAD // 32) // 64


@functools.partial(
    pl.kernel,
    mesh=_mesh,
    out_type=jax.ShapeDtypeStruct((2, 640, D), jnp.float32),
    scratch_types=[
        pltpu.VMEM((PJ, 64), jnp.int32),
        pltpu.VMEM((64, D), jnp.float32),
        pltpu.VMEM((64, D), jnp.float32),
        pltpu.VMEM((64, D), jnp.float32),
        pltpu.VMEM((64, D), jnp.float32),
        pltpu.VMEM((40, D), jnp.float32),
        pltpu.VMEM_SHARED((640, D), jnp.float32),
        pltpu.SemaphoreType.DMA,
        pltpu.SemaphoreType.DMA,
    ],
    compiler_params=_SC_PARAMS,
)
def _pool_kernel(a_hbm, b2_hbm, out, idxb, rb0, rb1, rb2, rb3, zbuf2, acc,
                 semG, semS):
    cid = lax.axis_index("c")
    sid = lax.axis_index("s")
    _zero_vmem2(zbuf2, 40, D)
    pltpu.sync_copy(zbuf2, acc.at[pl.ds(sid * 40, 40)])
    plsc.subcore_barrier()
    base0 = cid * (NPAD // 2) + sid * (NPAD // 32)
    RB = [rb0, rb1, rb2, rb3]
    pltpu.sync_copy(b2_hbm.at[pl.ds(cid * (PJ * 16) + sid * PJ, PJ)], idxb)

    def wait_g(b):
        pltpu.make_async_copy(a_hbm.at[pl.ds(0, 64)], RB[b], semG).wait()

    def wait_s(b):
        pltpu.make_async_copy(RB[b], acc.at[pl.ds(0, 64)], semS).wait()

    for j in range(PJ):
        b = j % 4
        if j >= 4:
            wait_s(b)
        pltpu.async_copy(a_hbm.at[pl.ds(base0 + j * 64, 64)], RB[b], semG)
        d = j - 2
        if d >= 0:
            bd = d % 4
            wait_g(bd)
            pltpu.async_copy(RB[bd], acc.at[idxb.at[d]], semS, add=True)
    for d in (PJ - 2, PJ - 1):
        bd = d % 4
        wait_g(bd)
        pltpu.async_copy(RB[bd], acc.at[idxb.at[d]], semS, add=True)
    for b in range(4):
        wait_s(b)
    plsc.subcore_barrier()
    pltpu.sync_copy(acc.at[pl.ds(sid * 40, 40)],
                    out.at[cid, pl.ds(sid * 40, 40)])


def _dis_of(ind_ref):
    ind = ind_ref[0, :] + ind_ref[1, :]
    return lax.rsqrt(ind + 2.0)[:, None]


def _tc_a_body(x_ref, ind_ref, We_ref, be_ref, W1_ref, a0_ref, g1_ref):
    dis = _dis_of(ind_ref)
    a0 = jnp.dot(jnp.log(x_ref[...] + 1.0), We_ref[...],
                 preferred_element_type=jnp.float32) + be_ref[...]
    a0_ref[...] = a0
    g1_ref[...] = dis * jnp.dot(a0, W1_ref[...],
                                preferred_element_type=jnp.float32)


def _tc_layer_body(a_ref, g_ref, s0, s1, s2, s3, ind_ref, W_ref, b_ref,
                   anew_ref, gnew_ref):
    dis = _dis_of(ind_ref)
    s = jnp.concatenate([s0[...], s1[...], s2[...], s3[...]], axis=1)
    anew = a_ref[...] + jnp.maximum(
        dis * (s + 2.0 * g_ref[...]) + b_ref[...], 0.0)
    anew_ref[...] = anew
    gnew_ref[...] = dis * jnp.dot(anew, W_ref[...],
                                  preferred_element_type=jnp.float32)


def _tc_final_body(a_ref, g_ref, s0, s1, s2, s3, ind_ref, b_ref, anew_ref):
    dis = _dis_of(ind_ref)
    s = jnp.concatenate([s0[...], s1[...], s2[...], s3[...]], axis=1)
    anew_ref[...] = a_ref[...] + jnp.maximum(
        dis * (s + 2.0 * g_ref[...]) + b_ref[...], 0.0)


def _row_spec(w=D):
    return pl.BlockSpec((BN, w), lambda i: (i, 0))


def _fix_spec(shape):
    return pl.BlockSpec(shape, lambda i: tuple(0 for _ in shape))


_IND_SPEC = pl.BlockSpec((2, BN), lambda i: (0, i))


def _tc_a(xp, ind2, We, be, W1):
    return pl.pallas_call(
        _tc_a_body,
        grid=(GRID,),
        in_specs=[_row_spec(16), _IND_SPEC, _fix_spec((16, D)),
                  _fix_spec((1, D)), _fix_spec((D, D))],
        out_specs=[_row_spec(), _row_spec()],
        out_shape=[jax.ShapeDtypeStruct((NPAD, D), jnp.float32)] * 2,
    )(xp, ind2, We, be, W1)


def _tc_layer(a, g, s4, ind2, W, b):
    return pl.pallas_call(
        _tc_layer_body,
        grid=(GRID,),
        in_specs=[_row_spec(), _row_spec(),
                  _row_spec(DQ), _row_spec(DQ), _row_spec(DQ), _row_spec(DQ),
                  _IND_SPEC, _fix_spec((D, D)), _fix_spec((1, D))],
        out_specs=[_row_spec(), _row_spec()],
        out_shape=[jax.ShapeDtypeStruct((NPAD, D), jnp.float32)] * 2,
    )(a, g, s4[0], s4[1], s4[2], s4[3], ind2, W, b)


def _tc_final(a, g, s4, ind2, b):
    return pl.pallas_call(
        _tc_final_body,
        grid=(GRID,),
        in_specs=[_row_spec(), _row_spec(),
                  _row_spec(DQ), _row_spec(DQ), _row_spec(DQ), _row_spec(DQ),
                  _IND_SPEC, _fix_spec((1, D))],
        out_specs=[_row_spec()],
        out_shape=[jax.ShapeDtypeStruct((NPAD, D), jnp.float32)],
    )(a, g, s4[0], s4[1], s4[2], s4[3], ind2, b)[0]


def kernel(x, edge_index, batch, W_exp, b_exp, Ws, bs):
    L = Ws.shape[0]
    r = edge_index[0]
    c = edge_index[1]
    xp = jnp.pad(jnp.asarray(x, jnp.float32), ((0, NPAD - N), (0, 5)))
    batchp = jnp.pad(batch.astype(jnp.int32), (0, NPAD - N),
                     constant_values=G)
    j = jnp.arange(EPADX - E, dtype=jnp.int32)
    rfull = jnp.concatenate([r, j % 4096])
    cflat = jnp.concatenate([c, NPAD + (j % 8)])
    r4q3 = ((rfull * 4)[None, :]
            + jnp.arange(4, dtype=jnp.int32)[:, None]).reshape(
                4, EPADX_ROWS, CH)
    c2 = cflat.reshape(EPADX_ROWS, CH)
    Wep = jnp.pad(jnp.asarray(W_exp, jnp.float32), ((0, 5), (0, 0)))

    ind2 = _deg_kernel(c2)
    a, g = _tc_a(xp, ind2, Wep, b_exp.reshape(1, D), Ws[0])
    for i in range(1, L + 1):
        s4 = _prop_kernel(g.reshape(NPAD * 4, DQ), r4q3, c2)
        if i < L:
            a, g = _tc_layer(a, g, s4, ind2, Ws[i], bs[i - 1].reshape(1, D))
        else:
            a = _tc_final(a, g, s4, ind2, bs[L - 1].reshape(1, D))
    parts = _pool_kernel(a, batchp.reshape(NPAD // 64, 64))
    return parts[0, :G] + parts[1, :G]
